# baseline (speedup 1.0000x reference)
import jax
import jax.numpy as jnp
from jax import lax
from jax.experimental import pallas as pl
from jax.experimental.pallas import tpu as pltpu
from jax.experimental.pallas import tpu_sc as plsc

NUM_ITEMS = 1000000
NUM_USERS = 100000
D = 32
B = 16384
LR = 0.01
WD = 0.0001
SCALE = 1.0 - WD

DH = D // 2
NS = 16
C = B // NS
SUB = 128
NSUB = C // SUB
IDX_ROWS = B // SUB


def _scale_body(x_ref, o_ref):
    o_ref[...] = x_ref[...] * SCALE


def _tc_scale(xt, cols_per_block):
    d, n = xt.shape
    return pl.pallas_call(
        _scale_body,
        grid=(pl.cdiv(n, cols_per_block),),
        in_specs=[pl.BlockSpec((d, cols_per_block), lambda i: (0, i))],
        out_specs=pl.BlockSpec((d, cols_per_block), lambda i: (0, i)),
        out_shape=jax.ShapeDtypeStruct((d, n), xt.dtype),
    )(xt)


def _sc_body(item_t, user_t, igt, ugt, ii, ui,
             pos_sh, acc_sh, cnt_sh,
             idx_v, slot_v, pid_v, ev, gbuf, svc, cg, inv_v, zrow, ones_v):
    c = lax.axis_index("c")
    s = lax.axis_index("s")
    iota16 = lax.iota(jnp.int32, 16)
    zeros16 = jnp.zeros((16,), jnp.float32)
    ones16 = jnp.full((16,), 1.0, jnp.float32)

    for k in range(SUB // 16):
        zrow[pl.ds(16 * k, 16)] = zeros16
        ones_v[pl.ds(16 * k, 16)] = ones16

    def run(tref, gref, iref):
        base = s * C
        pltpu.sync_copy(iref.at[pl.ds(s * NSUB, NSUB)], idx_v)

        def _pid(i, _):
            pid_v[pl.ds(16 * i, 16)] = iota16 + (base + 16 * i)
            return 0
        lax.fori_loop(0, C // 16, _pid, 0, unroll=False)

        def _rep(j, _):
            pltpu.sync_copy(pid_v.at[pl.ds(j * SUB, SUB)],
                            pos_sh.at[idx_v.at[j]])
            pltpu.sync_copy(zrow, cnt_sh.at[pl.ds(base + j * SUB, SUB)])
            return 0
        lax.fori_loop(0, NSUB, _rep, 0, unroll=False)
        plsc.subcore_barrier()

        def _slots(j, _):
            pltpu.sync_copy(pos_sh.at[idx_v.at[j]], slot_v.at[j])
            pltpu.sync_copy(ones_v, cnt_sh.at[slot_v.at[j]], add=True)
            return 0
        lax.fori_loop(0, NSUB, _slots, 0, unroll=False)
        plsc.subcore_barrier()

        def _cg(j, _):
            pltpu.sync_copy(cnt_sh.at[slot_v.at[j]], cg.at[pl.ds(j * SUB, SUB)])
            return 0
        lax.fori_loop(0, NSUB, _cg, 0, unroll=False)

        def _inv(i, _):
            sl = pl.ds(16 * i, 16)
            inv_v[sl] = LR / cg[sl]
            return 0
        lax.fori_loop(0, C // 16, _inv, 0, unroll=False)

        for half in range(2):
            d0 = half * DH

            def _zero(i, _):
                d = i // NSUB
                j = i % NSUB
                pltpu.sync_copy(zrow,
                                acc_sh.at[d, pl.ds(base + j * SUB, SUB)])
                return 0
            lax.fori_loop(0, DH * NSUB, _zero, 0, unroll=False)
            plsc.subcore_barrier()

            def _sadd(d, _):
                pltpu.sync_copy(gref.at[d0 + d, pl.ds(base, C)], gbuf)

                def _sa2(j, _):
                    pltpu.sync_copy(gbuf.at[pl.ds(j * SUB, SUB)],
                                    acc_sh.at[d].at[slot_v.at[j]], add=True)
                    return 0
                lax.fori_loop(0, NSUB, _sa2, 0, unroll=False)
                return 0
            lax.fori_loop(0, DH, _sadd, 0, unroll=False)
            plsc.subcore_barrier()

            def _gath(i, _):
                d = i // NSUB
                j = i % NSUB
                pltpu.sync_copy(tref.at[d0 + d].at[idx_v.at[j]],
                                ev.at[pl.ds(d * C + j * SUB, SUB)])
                pltpu.sync_copy(acc_sh.at[d].at[slot_v.at[j]], svc)

                def _comb(k, _):
                    sl = pl.ds(d * C + j * SUB + 16 * k, 16)
                    ev[sl] = (ev[sl]
                              - inv_v[pl.ds(j * SUB + 16 * k, 16)]
                              * svc[pl.ds(16 * k, 16)])
                    return 0
                lax.fori_loop(0, SUB // 16, _comb, 0, unroll=False)
                return 0
            lax.fori_loop(0, DH * NSUB, _gath, 0, unroll=False)
            plsc.subcore_barrier()

            def _wb(i, _):
                d = i // NSUB
                j = i % NSUB
                pltpu.sync_copy(ev.at[pl.ds(d * C + j * SUB, SUB)],
                                tref.at[d0 + d].at[idx_v.at[j]])
                return 0
            lax.fori_loop(0, DH * NSUB, _wb, 0, unroll=False)
            plsc.subcore_barrier()

    @pl.when(c == 0)
    def _():
        run(item_t, igt, ii)

    @pl.when(c == 1)
    def _():
        run(user_t, ugt, ui)


def kernel(item_emb, user_emb, item_grad, user_grad, item_idx, user_idx):
    ii = jnp.reshape(item_idx.astype(jnp.int32), (IDX_ROWS, SUB))
    ui = jnp.reshape(user_idx.astype(jnp.int32), (IDX_ROWS, SUB))
    scaled_i = _tc_scale(jnp.transpose(item_emb), 65536)
    scaled_u = _tc_scale(jnp.transpose(user_emb), 65536)
    ri = jax.new_ref(scaled_i)
    ru = jax.new_ref(scaled_u)
    mesh = plsc.VectorSubcoreMesh(core_axis_name="c", subcore_axis_name="s")
    fix = pl.kernel(
        _sc_body,
        out_type=(),
        mesh=mesh,
        compiler_params=pltpu.CompilerParams(
            needs_layout_passes=False, use_tc_tiling_on_sc=False),
        scratch_types=[
            pltpu.VMEM_SHARED((NUM_ITEMS,), jnp.int32),
            pltpu.VMEM_SHARED((DH, B), jnp.float32),
            pltpu.VMEM_SHARED((B,), jnp.float32),
            pltpu.VMEM((NSUB, SUB), jnp.int32),
            pltpu.VMEM((NSUB, SUB), jnp.int32),
            pltpu.VMEM((C,), jnp.int32),
            pltpu.VMEM((DH * C,), jnp.float32),
            pltpu.VMEM((C,), jnp.float32),
            pltpu.VMEM((SUB,), jnp.float32),
            pltpu.VMEM((C,), jnp.float32),
            pltpu.VMEM((C,), jnp.float32),
            pltpu.VMEM((SUB,), jnp.float32),
            pltpu.VMEM((SUB,), jnp.float32),
        ],
    )
    fix(ri, ru, jnp.transpose(item_grad), jnp.transpose(user_grad), ii, ui)
    return jnp.transpose(ri[...]), jnp.transpose(ru[...])


# speedup vs baseline: 4.2169x; 4.2169x over previous
import jax
import jax.numpy as jnp
from jax import lax
from jax.experimental import pallas as pl
from jax.experimental.pallas import tpu as pltpu
from jax.experimental.pallas import tpu_sc as plsc

NUM_ITEMS = 1000000
NUM_USERS = 100000
D = 32
B = 16384
LR = 0.01
WD = 0.0001
SCALE = 1.0 - WD

NS = 16
NW = 32
C = B // NS
SUB = 128
NSUB = C // SUB
IDX_ROWS = B // SUB

HALF_ITEMS = NUM_ITEMS // 2
TRASH = HALF_ITEMS
POS_SIZE = HALF_ITEMS + 8

ITEM_ROWS_W = NUM_ITEMS // NW
USER_ROWS_W = NUM_USERS // NW
ICHUNK = 1250
UCHUNK = 625

_SC_PARAMS = pltpu.CompilerParams(
    needs_layout_passes=False, use_tc_tiling_on_sc=False)


def _dense_body(item_in, user_in, item_out, user_out, buf):
    c = lax.axis_index("c")
    s = lax.axis_index("s")
    wid = s * 2 + c

    def _scale_chunk(tin, tout, row0, nrows):
        pltpu.sync_copy(tin.at[pl.ds(row0, nrows)], buf.at[pl.ds(0, nrows)])

        def _mul(i, _):
            r = i // 2
            h = (i % 2) * 16
            buf[r, pl.ds(h, 16)] = buf[r, pl.ds(h, 16)] * SCALE
            return 0
        lax.fori_loop(0, nrows * 2, _mul, 0, unroll=8)
        pltpu.sync_copy(buf.at[pl.ds(0, nrows)], tout.at[pl.ds(row0, nrows)])

    def _item(k, _):
        _scale_chunk(item_in, item_out, wid * ITEM_ROWS_W + k * ICHUNK, ICHUNK)
        return 0
    lax.fori_loop(0, ITEM_ROWS_W // ICHUNK, _item, 0, unroll=False)

    def _user(k, _):
        _scale_chunk(user_in, user_out, wid * USER_ROWS_W + k * UCHUNK, UCHUNK)
        return 0
    lax.fori_loop(0, USER_ROWS_W // UCHUNK, _user, 0, unroll=False)


def _sparse_body(item_t, user_t, item_e, user_e, ig, ug, ii2, ui2,
                 pos_sh, acc_sh, cnt_sh,
                 idx_v, sidx_v, widx_v, slot_v, pid_v, cg, inv_v,
                 gb, eb, sb, wb, zb32, zrow, ones_v):
    c = lax.axis_index("c")
    s = lax.axis_index("s")
    base = s * C
    iota16 = lax.iota(jnp.int32, 16)
    zeros16 = jnp.zeros((16,), jnp.float32)
    ones16 = jnp.full((16,), 1.0, jnp.float32)

    for k in range(SUB // 16):
        zrow[pl.ds(16 * k, 16)] = zeros16
        ones_v[pl.ds(16 * k, 16)] = ones16

    def _zb(i, _):
        zb32[i // 2, pl.ds((i % 2) * 16, 16)] = zeros16
        return 0
    lax.fori_loop(0, SUB * 2, _zb, 0, unroll=False)

    def run(tref, eref, gref, iref2, lo, hi, nrows):
        pltpu.sync_copy(iref2.at[pl.ds(s * NSUB, NSUB)], idx_v)

        def _prep(i, _):
            v = idx_v[i // 8, pl.ds((i % 8) * 16, 16)]
            act = (v >= lo) & (v < hi)
            sidx_v[i // 8, pl.ds((i % 8) * 16, 16)] = jnp.where(
                act, v - lo, jnp.full((16,), TRASH, jnp.int32))
            widx_v[i // 8, pl.ds((i % 8) * 16, 16)] = jnp.where(
                act, v, nrows + (iota16 & 7))
            pid_v[pl.ds(16 * i, 16)] = iota16 + (base + 16 * i)
            return 0
        lax.fori_loop(0, C // 16, _prep, 0, unroll=False)

        def _rep(j, _):
            pltpu.sync_copy(pid_v.at[pl.ds(j * SUB, SUB)],
                            pos_sh.at[sidx_v.at[j]])
            pltpu.sync_copy(zrow, cnt_sh.at[pl.ds(base + j * SUB, SUB)])
            return 0
        lax.fori_loop(0, NSUB, _rep, 0, unroll=False)
        plsc.subcore_barrier()

        def _slots(j, _):
            pltpu.sync_copy(pos_sh.at[sidx_v.at[j]], slot_v.at[j])
            pltpu.sync_copy(ones_v, cnt_sh.at[slot_v.at[j]], add=True)
            pltpu.sync_copy(zb32, acc_sh.at[pl.ds(base + j * SUB, SUB)])
            return 0
        lax.fori_loop(0, NSUB, _slots, 0, unroll=False)
        plsc.subcore_barrier()

        def _cgt(j, _):
            pltpu.sync_copy(cnt_sh.at[slot_v.at[j]], cg.at[pl.ds(j * SUB, SUB)])
            return 0
        lax.fori_loop(0, NSUB, _cgt, 0, unroll=False)

        def _inv(i, _):
            sl = pl.ds(16 * i, 16)
            v = idx_v[i // 8, pl.ds((i % 8) * 16, 16)]
            act = (v >= lo) & (v < hi)
            inv_v[sl] = jnp.where(act, LR / cg[sl], zeros16)
            return 0
        lax.fori_loop(0, C // 16, _inv, 0, unroll=False)

        def _sadd(j, _):
            pltpu.sync_copy(gref.at[pl.ds(base + j * SUB, SUB)], gb)
            pltpu.sync_copy(gb, acc_sh.at[slot_v.at[j]], add=True)
            return 0
        lax.fori_loop(0, NSUB, _sadd, 0, unroll=False)
        plsc.subcore_barrier()

        def _comb(j, _):
            pltpu.sync_copy(eref.at[idx_v.at[j]], eb)
            pltpu.sync_copy(acc_sh.at[slot_v.at[j]], sb)

            def _row(r, _):
                f = plsc.load_gather(
                    inv_v, [jnp.full((16,), 0, jnp.int32) + (j * SUB + r)])
                for k in range(2):
                    sl = pl.ds(16 * k, 16)
                    wb[r, sl] = eb[r, sl] * SCALE - f * sb[r, sl]
                return 0
            lax.fori_loop(0, SUB, _row, 0, unroll=False)
            pltpu.sync_copy(wb, tref.at[widx_v.at[j]])
            return 0
        lax.fori_loop(0, NSUB, _comb, 0, unroll=False)
        plsc.subcore_barrier()

    @pl.when(c == 0)
    def _():
        run(item_t, item_e, ig, ii2, 0, HALF_ITEMS, NUM_ITEMS)
        run(item_t, item_e, ig, ii2, HALF_ITEMS, NUM_ITEMS, NUM_ITEMS)

    @pl.when(c == 1)
    def _():
        run(user_t, user_e, ug, ui2, 0, NUM_USERS, NUM_USERS)


def kernel(item_emb, user_emb, item_grad, user_grad, item_idx, user_idx):
    iif = item_idx.astype(jnp.int32)
    uif = user_idx.astype(jnp.int32)
    ii2 = jnp.reshape(iif, (IDX_ROWS, SUB))
    ui2 = jnp.reshape(uif, (IDX_ROWS, SUB))

    mesh = plsc.VectorSubcoreMesh(core_axis_name="c", subcore_axis_name="s")
    dense = pl.kernel(
        _dense_body,
        out_type=(jax.ShapeDtypeStruct((NUM_ITEMS + 8, D), jnp.float32),
                  jax.ShapeDtypeStruct((NUM_USERS + 8, D), jnp.float32)),
        mesh=mesh,
        compiler_params=_SC_PARAMS,
        scratch_types=[pltpu.VMEM((ICHUNK, D), jnp.float32)],
    )
    scaled_i, scaled_u = dense(item_emb, user_emb)

    ri = jax.new_ref(scaled_i)
    ru = jax.new_ref(scaled_u)
    fix = pl.kernel(
        _sparse_body,
        out_type=(),
        mesh=mesh,
        compiler_params=_SC_PARAMS,
        scratch_types=[
            pltpu.VMEM_SHARED((POS_SIZE,), jnp.int32),
            pltpu.VMEM_SHARED((B, D), jnp.float32),
            pltpu.VMEM_SHARED((B,), jnp.float32),
            pltpu.VMEM((NSUB, SUB), jnp.int32),
            pltpu.VMEM((NSUB, SUB), jnp.int32),
            pltpu.VMEM((NSUB, SUB), jnp.int32),
            pltpu.VMEM((NSUB, SUB), jnp.int32),
            pltpu.VMEM((C,), jnp.int32),
            pltpu.VMEM((C,), jnp.float32),
            pltpu.VMEM((C,), jnp.float32),
            pltpu.VMEM((SUB, D), jnp.float32),
            pltpu.VMEM((SUB, D), jnp.float32),
            pltpu.VMEM((SUB, D), jnp.float32),
            pltpu.VMEM((SUB, D), jnp.float32),
            pltpu.VMEM((SUB, D), jnp.float32),
            pltpu.VMEM((SUB,), jnp.float32),
            pltpu.VMEM((SUB,), jnp.float32),
        ],
    )
    fix(ri, ru, item_emb, user_emb, item_grad, user_grad, ii2, ui2)
    return ri[...][:NUM_ITEMS], ru[...][:NUM_USERS]
